# baseline (speedup 1.0000x reference)
import functools

import jax
import jax.numpy as jnp
from jax import lax
from jax.experimental import pallas as pl
from jax.experimental.pallas import tpu as pltpu
from jax.experimental.pallas import tpu_sc as plsc

HIDDEN = 128
NUM_GRAPHS = 125
VARS_PER_GRAPH = 2000
NUM_VARS = NUM_GRAPHS * VARS_PER_GRAPH
NUM_LITS = 2 * NUM_VARS

NUM_WORKERS = 32
TC_GRAPHS = 56
SC_GRAPHS = NUM_GRAPHS - TC_GRAPHS
GRAPHS_PER_WORKER = -(-SC_GRAPHS // NUM_WORKERS)
CHUNK = 250
CHUNKS_PER_GRAPH = (2 * VARS_PER_GRAPH) // CHUNK
LANES = 16
VREGS_PER_ROW = HIDDEN // LANES


def _sc_segment_sum_body(l_hbm, out_hbm, buf0, buf1, row_v, sem0, sem1):
    wid = lax.axis_index("s") * 2 + lax.axis_index("c")
    bufs = (buf0, buf1)
    sems = (sem0, sem1)

    for gi in range(GRAPHS_PER_WORKER):
        g = TC_GRAPHS + wid + NUM_WORKERS * gi

        @pl.when(g < NUM_GRAPHS)
        def _():
            def chunk_src(i):
                half = i // (CHUNKS_PER_GRAPH // 2)
                c = i % (CHUNKS_PER_GRAPH // 2)
                start = (half * NUM_VARS + g * VARS_PER_GRAPH
                         + c * CHUNK) * HIDDEN
                return l_hbm.at[pl.ds(start, CHUNK * HIDDEN)]

            copies = [None] * CHUNKS_PER_GRAPH
            copies[0] = pltpu.async_copy(chunk_src(0), bufs[0], sems[0])
            acc = tuple(jnp.zeros((LANES,), jnp.float32)
                        for _ in range(VREGS_PER_ROW))
            for i in range(CHUNKS_PER_GRAPH):
                if i + 1 < CHUNKS_PER_GRAPH:
                    copies[i + 1] = pltpu.async_copy(
                        chunk_src(i + 1), bufs[(i + 1) % 2], sems[(i + 1) % 2])
                copies[i].wait()
                buf = bufs[i % 2]

                def body(r, a):
                    base = r * HIDDEN
                    return tuple(
                        a[j] + buf[pl.ds(base + LANES * j, LANES)]
                        for j in range(VREGS_PER_ROW))

                acc = lax.fori_loop(0, CHUNK, body, acc, unroll=2)

            for j in range(VREGS_PER_ROW):
                row_v[pl.ds(LANES * j, LANES)] = acc[j]
            pltpu.sync_copy(row_v, out_hbm.at[pl.ds(g * HIDDEN, HIDDEN)])


@functools.partial(
    pl.kernel,
    out_type=jax.ShapeDtypeStruct((NUM_GRAPHS * HIDDEN,), jnp.float32),
    mesh=plsc.VectorSubcoreMesh(core_axis_name="c", subcore_axis_name="s"),
    scratch_types=[
        pltpu.VMEM((CHUNK * HIDDEN,), jnp.float32),
        pltpu.VMEM((CHUNK * HIDDEN,), jnp.float32),
        pltpu.VMEM((HIDDEN,), jnp.float32),
        pltpu.SemaphoreType.DMA,
        pltpu.SemaphoreType.DMA,
    ],
)
def _sc_segment_sum(l_hbm, out_hbm, buf0, buf1, row_v, sem0, sem1):
    _sc_segment_sum_body(l_hbm, out_hbm, buf0, buf1, row_v, sem0, sem1)


def _tc_sum_kernel(x_ref, o_ref):
    o_ref[...] = jnp.sum(x_ref[0, 0] + x_ref[1, 0], axis=0,
                         keepdims=True)[None]


def _tc_graph_sums(l4):
    return pl.pallas_call(
        _tc_sum_kernel,
        grid=(TC_GRAPHS,),
        in_specs=[pl.BlockSpec((2, 1, VARS_PER_GRAPH, HIDDEN),
                               lambda g: (0, g, 0, 0))],
        out_specs=pl.BlockSpec((1, 1, HIDDEN), lambda g: (g, 0, 0)),
        out_shape=jax.ShapeDtypeStruct((TC_GRAPHS, 1, HIDDEN), jnp.float32),
    )(l4)


def _mlp_kernel(tc_ref, sc_ref, nv_ref, w1_ref, b1_ref, w2_ref, b2_ref,
                w3t_ref, b3_ref, o_ref):
    sums = jnp.concatenate(
        [tc_ref[:, 0, :], sc_ref[TC_GRAPHS:, :]], axis=0)
    x = sums * (0.5 / nv_ref[...])
    h = jnp.maximum(
        jnp.dot(x, w1_ref[...], preferred_element_type=jnp.float32)
        + b1_ref[...], 0.0)
    h = jnp.maximum(
        jnp.dot(h, w2_ref[...], preferred_element_type=jnp.float32)
        + b2_ref[...], 0.0)
    o = jnp.sum(h * w3t_ref[...], axis=1, keepdims=True) + b3_ref[...]
    o_ref[...] = jax.nn.sigmoid(o)


def kernel(l_embedding, num_variable, W1, b1, W2, b2, W3, b3):
    sc_sums = _sc_segment_sum(l_embedding.reshape(NUM_LITS * HIDDEN))
    tc_sums = _tc_graph_sums(
        l_embedding.reshape(2, NUM_GRAPHS, VARS_PER_GRAPH, HIDDEN))
    nv = num_variable.astype(jnp.float32).reshape(NUM_GRAPHS, 1)
    out = pl.pallas_call(
        _mlp_kernel,
        out_shape=jax.ShapeDtypeStruct((NUM_GRAPHS, 1), jnp.float32),
    )(tc_sums, sc_sums.reshape(NUM_GRAPHS, HIDDEN), nv, W1,
      b1.reshape(1, HIDDEN), W2, b2.reshape(1, HIDDEN),
      W3.reshape(1, HIDDEN), b3.reshape(1, 1))
    return out.reshape(NUM_GRAPHS)


# speedup vs baseline: 1.2269x; 1.2269x over previous
import functools

import jax
import jax.numpy as jnp
from jax import lax
from jax.experimental import pallas as pl
from jax.experimental.pallas import tpu as pltpu
from jax.experimental.pallas import tpu_sc as plsc

HIDDEN = 128
NUM_GRAPHS = 125
VARS_PER_GRAPH = 2000
NUM_VARS = NUM_GRAPHS * VARS_PER_GRAPH
NUM_LITS = 2 * NUM_VARS

NUM_WORKERS = 32
TC_GRAPHS = 61
SC_GRAPHS = NUM_GRAPHS - TC_GRAPHS
GRAPHS_PER_WORKER = -(-SC_GRAPHS // NUM_WORKERS)
CHUNK = 250
CHUNKS_PER_GRAPH = (2 * VARS_PER_GRAPH) // CHUNK
LANES = 16
VREGS_PER_ROW = HIDDEN // LANES


def _sc_segment_sum_body(l_hbm, out_hbm, buf0, buf1, row_v, sem0, sem1):
    wid = lax.axis_index("s") * 2 + lax.axis_index("c")
    bufs = (buf0, buf1)
    sems = (sem0, sem1)

    for gi in range(GRAPHS_PER_WORKER):
        g = TC_GRAPHS + wid + NUM_WORKERS * gi

        @pl.when(g < NUM_GRAPHS)
        def _():
            def chunk_src(i):
                half = i // (CHUNKS_PER_GRAPH // 2)
                c = i % (CHUNKS_PER_GRAPH // 2)
                start = (half * NUM_VARS + g * VARS_PER_GRAPH
                         + c * CHUNK) * HIDDEN
                return l_hbm.at[pl.ds(start, CHUNK * HIDDEN)]

            copies = [None] * CHUNKS_PER_GRAPH
            copies[0] = pltpu.async_copy(chunk_src(0), bufs[0], sems[0])
            acc = tuple(jnp.zeros((LANES,), jnp.float32)
                        for _ in range(VREGS_PER_ROW))
            for i in range(CHUNKS_PER_GRAPH):
                if i + 1 < CHUNKS_PER_GRAPH:
                    copies[i + 1] = pltpu.async_copy(
                        chunk_src(i + 1), bufs[(i + 1) % 2], sems[(i + 1) % 2])
                copies[i].wait()
                buf = bufs[i % 2]

                def body(r, a):
                    base = r * HIDDEN
                    return tuple(
                        a[j] + buf[pl.ds(base + LANES * j, LANES)]
                        for j in range(VREGS_PER_ROW))

                acc = lax.fori_loop(0, CHUNK, body, acc, unroll=2)

            for j in range(VREGS_PER_ROW):
                row_v[pl.ds(LANES * j, LANES)] = acc[j]
            pltpu.sync_copy(row_v, out_hbm.at[pl.ds(g * HIDDEN, HIDDEN)])


@functools.partial(
    pl.kernel,
    out_type=jax.ShapeDtypeStruct((NUM_GRAPHS * HIDDEN,), jnp.float32),
    mesh=plsc.VectorSubcoreMesh(core_axis_name="c", subcore_axis_name="s"),
    scratch_types=[
        pltpu.VMEM((CHUNK * HIDDEN,), jnp.float32),
        pltpu.VMEM((CHUNK * HIDDEN,), jnp.float32),
        pltpu.VMEM((HIDDEN,), jnp.float32),
        pltpu.SemaphoreType.DMA,
        pltpu.SemaphoreType.DMA,
    ],
)
def _sc_segment_sum(l_hbm, out_hbm, buf0, buf1, row_v, sem0, sem1):
    _sc_segment_sum_body(l_hbm, out_hbm, buf0, buf1, row_v, sem0, sem1)


def _tc_sum_kernel(x_ref, o_ref):
    o_ref[...] = jnp.sum(x_ref[0, 0] + x_ref[1, 0], axis=0,
                         keepdims=True)[None]


def _tc_graph_sums(l4):
    return pl.pallas_call(
        _tc_sum_kernel,
        grid=(TC_GRAPHS,),
        in_specs=[pl.BlockSpec((2, 1, VARS_PER_GRAPH, HIDDEN),
                               lambda g: (0, g, 0, 0))],
        out_specs=pl.BlockSpec((1, 1, HIDDEN), lambda g: (g, 0, 0)),
        out_shape=jax.ShapeDtypeStruct((TC_GRAPHS, 1, HIDDEN), jnp.float32),
    )(l4)


def _mlp_kernel(tc_ref, sc_ref, nv_ref, w1_ref, b1_ref, w2_ref, b2_ref,
                w3t_ref, b3_ref, o_ref):
    sums = jnp.concatenate(
        [tc_ref[:, 0, :], sc_ref[TC_GRAPHS:, :]], axis=0)
    x = sums * (0.5 / nv_ref[...])
    h = jnp.maximum(
        jnp.dot(x, w1_ref[...], preferred_element_type=jnp.float32)
        + b1_ref[...], 0.0)
    h = jnp.maximum(
        jnp.dot(h, w2_ref[...], preferred_element_type=jnp.float32)
        + b2_ref[...], 0.0)
    o = jnp.sum(h * w3t_ref[...], axis=1, keepdims=True) + b3_ref[...]
    o_ref[...] = jax.nn.sigmoid(o)


def kernel(l_embedding, num_variable, W1, b1, W2, b2, W3, b3):
    sc_sums = _sc_segment_sum(l_embedding.reshape(NUM_LITS * HIDDEN))
    tc_sums = _tc_graph_sums(
        l_embedding.reshape(2, NUM_GRAPHS, VARS_PER_GRAPH, HIDDEN))
    nv = num_variable.astype(jnp.float32).reshape(NUM_GRAPHS, 1)
    out = pl.pallas_call(
        _mlp_kernel,
        out_shape=jax.ShapeDtypeStruct((NUM_GRAPHS, 1), jnp.float32),
    )(tc_sums, sc_sums.reshape(NUM_GRAPHS, HIDDEN), nv, W1,
      b1.reshape(1, HIDDEN), W2, b2.reshape(1, HIDDEN),
      W3.reshape(1, HIDDEN), b3.reshape(1, 1))
    return out.reshape(NUM_GRAPHS)
